# baseline (speedup 1.0000x reference)
import functools
import math

import jax
import jax.numpy as jnp
from jax import lax
from jax.experimental import pallas as pl
from jax.experimental.pallas import tpu as pltpu
from jax.experimental.pallas import tpu_sc as plsc

_S = 64.0
_MARGIN = 0.5
_COS_M = math.cos(_MARGIN)
_SIN_M = math.sin(_MARGIN)

_RB = 256
_CB = 2048

_NC = 2
_NS = 16
_NW = _NC * _NS
_L = 16


def _sqrt16(a):
    s = 0.5 * (a + 1.0)
    for _ in range(20):
        s = 0.5 * (s + a / jnp.maximum(s, 1e-30))
    return s


def _sc_margin_body(logits_hbm, lab_hbm, out_hbm, lab_v, gat_v, cor_v, sem, *, per_w):
    wid = lax.axis_index("s") * _NC + lax.axis_index("c")
    base = wid * per_w
    lane_iota = lax.iota(jnp.int32, _L)
    pltpu.sync_copy(lab_hbm.at[pl.ds(base, per_w)], lab_v)
    for w in range(per_w // _L):
        lab = lab_v[pl.ds(w * _L, _L)]
        safe = jnp.where(lab < 0, 0, lab)
        cs = [safe[l] for l in range(_L)]
        handles = []
        for l in range(_L):
            r = base + w * _L + l
            c = cs[l]
            handles.append(
                pltpu.async_copy(
                    logits_hbm.at[
                        pl.ds(pl.multiple_of(r & ~7, 8), 8),
                        pl.ds(pl.multiple_of(c & ~127, 128), 128),
                    ],
                    gat_v.at[l],
                    sem,
                )
            )
        for h in handles:
            h.wait()
        acc = jnp.zeros((_L,), jnp.float32)
        for l in range(_L):
            k = w * _L + l
            c = cs[l]
            cc0 = (c & 127) & ~15
            v = gat_v[l, k & 7, pl.ds(pl.multiple_of(cc0, 8), _L)]
            idx = jnp.broadcast_to(c & 15, (_L,))[:, None]
            t16 = lax.gather(
                v, idx,
                dimension_numbers=lax.GatherDimensionNumbers(
                    offset_dims=(), collapsed_slice_dims=(0,), start_index_map=(0,)),
                slice_sizes=(1,),
                mode=lax.GatherScatterMode.PROMISE_IN_BOUNDS,
            )
            acc = jnp.where(lane_iota == l, t16, acc)
        sin_theta = _sqrt16(jnp.maximum((1.0 - acc) * (1.0 + acc), 0.0))
        cor_v[pl.ds(w * _L, _L)] = (_COS_M * acc - _SIN_M * sin_theta) * _S
    pltpu.sync_copy(cor_v, out_hbm.at[pl.ds(base, per_w)])


def _sc_margin(logits, labels):
    rows = labels.shape[0]
    per_w = rows // _NW
    mesh = plsc.VectorSubcoreMesh(core_axis_name="c", subcore_axis_name="s")
    return pl.kernel(
        functools.partial(_sc_margin_body, per_w=per_w),
        out_type=jax.ShapeDtypeStruct((rows,), jnp.float32),
        mesh=mesh,
        scratch_types=[
            pltpu.VMEM((per_w,), jnp.int32),
            pltpu.VMEM((_L, 8, 128), jnp.float32),
            pltpu.VMEM((per_w,), jnp.float32),
            pltpu.SemaphoreType.DMA,
        ],
    )(logits, labels)


def _dense_body(lab_ref, cor_ref, x_ref, out_ref, *, cb):
    j = pl.program_id(1)
    x = x_ref[...]
    lab = lab_ref[...]
    cor = cor_ref[...]
    cols = j * cb + jax.lax.broadcasted_iota(jnp.int32, x.shape, 1)
    out_ref[...] = jnp.where(cols == lab, cor, x * _S)


def kernel(logits, labels):
    rows, cols = logits.shape
    corrected = _sc_margin(logits, labels)
    lab2 = labels.reshape(rows, 1)
    cor2 = corrected.reshape(rows, 1)
    grid = (rows // _RB, pl.cdiv(cols, _CB))
    return pl.pallas_call(
        functools.partial(_dense_body, cb=_CB),
        grid=grid,
        in_specs=[
            pl.BlockSpec((_RB, 1), lambda i, j: (i, 0)),
            pl.BlockSpec((_RB, 1), lambda i, j: (i, 0)),
            pl.BlockSpec((_RB, _CB), lambda i, j: (i, j)),
        ],
        out_specs=pl.BlockSpec((_RB, _CB), lambda i, j: (i, j)),
        out_shape=jax.ShapeDtypeStruct((rows, cols), jnp.float32),
    )(lab2, cor2, logits)


# speedup vs baseline: 1.0295x; 1.0295x over previous
import functools
import math

import jax
import jax.numpy as jnp
from jax import lax
from jax.experimental import pallas as pl
from jax.experimental.pallas import tpu as pltpu
from jax.experimental.pallas import tpu_sc as plsc

_S = 64.0
_MARGIN = 0.5
_COS_M = math.cos(_MARGIN)
_SIN_M = math.sin(_MARGIN)

_RB = 512
_CB = 4096

_NC = 2
_NS = 16
_NW = _NC * _NS
_L = 16


def _sqrt16(a):
    s = 0.5 * (a + 1.0)
    for _ in range(20):
        s = 0.5 * (s + a / jnp.maximum(s, 1e-30))
    return s


def _sc_margin_body(logits_hbm, lab_hbm, out_hbm, lab_v, gat_v, cor_v, sem, *, per_w):
    wid = lax.axis_index("s") * _NC + lax.axis_index("c")
    base = wid * per_w
    lane_iota = lax.iota(jnp.int32, _L)
    pltpu.sync_copy(lab_hbm.at[pl.ds(base, per_w)], lab_v)
    for w in range(per_w // _L):
        lab = lab_v[pl.ds(w * _L, _L)]
        safe = jnp.where(lab < 0, 0, lab)
        cs = [safe[l] for l in range(_L)]
        handles = []
        for l in range(_L):
            r = base + w * _L + l
            c = cs[l]
            handles.append(
                pltpu.async_copy(
                    logits_hbm.at[
                        pl.ds(pl.multiple_of(r & ~7, 8), 8),
                        pl.ds(pl.multiple_of(c & ~127, 128), 128),
                    ],
                    gat_v.at[l],
                    sem,
                )
            )
        for h in handles:
            h.wait()
        acc = jnp.zeros((_L,), jnp.float32)
        for l in range(_L):
            k = w * _L + l
            c = cs[l]
            cc0 = (c & 127) & ~15
            v = gat_v[l, k & 7, pl.ds(pl.multiple_of(cc0, 8), _L)]
            idx = jnp.broadcast_to(c & 15, (_L,))[:, None]
            t16 = lax.gather(
                v, idx,
                dimension_numbers=lax.GatherDimensionNumbers(
                    offset_dims=(), collapsed_slice_dims=(0,), start_index_map=(0,)),
                slice_sizes=(1,),
                mode=lax.GatherScatterMode.PROMISE_IN_BOUNDS,
            )
            acc = jnp.where(lane_iota == l, t16, acc)
        sin_theta = _sqrt16(jnp.maximum((1.0 - acc) * (1.0 + acc), 0.0))
        cor_v[pl.ds(w * _L, _L)] = (_COS_M * acc - _SIN_M * sin_theta) * _S
    pltpu.sync_copy(cor_v, out_hbm.at[pl.ds(base, per_w)])


def _sc_margin(logits, labels):
    rows = labels.shape[0]
    per_w = rows // _NW
    mesh = plsc.VectorSubcoreMesh(core_axis_name="c", subcore_axis_name="s")
    return pl.kernel(
        functools.partial(_sc_margin_body, per_w=per_w),
        out_type=jax.ShapeDtypeStruct((rows,), jnp.float32),
        mesh=mesh,
        scratch_types=[
            pltpu.VMEM((per_w,), jnp.int32),
            pltpu.VMEM((_L, 8, 128), jnp.float32),
            pltpu.VMEM((per_w,), jnp.float32),
            pltpu.SemaphoreType.DMA,
        ],
    )(logits, labels)


def _dense_body(lab_ref, cor_ref, x_ref, out_ref, *, cb):
    j = pl.program_id(1)
    x = x_ref[...]
    lab = lab_ref[...]
    cor = cor_ref[...]
    cols = j * cb + jax.lax.broadcasted_iota(jnp.int32, x.shape, 1)
    out_ref[...] = jnp.where(cols == lab, cor, x * _S)


def kernel(logits, labels):
    rows, cols = logits.shape
    corrected = _sc_margin(logits, labels)
    lab2 = labels.reshape(rows, 1)
    cor2 = corrected.reshape(rows, 1)
    grid = (rows // _RB, pl.cdiv(cols, _CB))
    return pl.pallas_call(
        functools.partial(_dense_body, cb=_CB),
        grid=grid,
        in_specs=[
            pl.BlockSpec((_RB, 1), lambda i, j: (i, 0)),
            pl.BlockSpec((_RB, 1), lambda i, j: (i, 0)),
            pl.BlockSpec((_RB, _CB), lambda i, j: (i, j)),
        ],
        out_specs=pl.BlockSpec((_RB, _CB), lambda i, j: (i, j)),
        out_shape=jax.ShapeDtypeStruct((rows, cols), jnp.float32),
    )(lab2, cor2, logits)
